# baseline (speedup 1.0000x reference)
import functools

import jax
import jax.numpy as jnp
from jax import lax
from jax.experimental import pallas as pl
from jax.experimental.pallas import tpu as pltpu
from jax.experimental.pallas import tpu_sc as plsc

N = 10000
E = 320000
D = 128
NC = 2
NS = 16
NW = NC * NS
K = 80
CH = 128
EPW = CH * K
PAD = NW * EPW - E
NP = 10112
RPT = NP // NS
NG = CH // 2

_mesh = plsc.VectorSubcoreMesh(core_axis_name="c", subcore_axis_name="s")


@functools.partial(
    pl.kernel,
    out_type=(
        jax.ShapeDtypeStruct((NC, NP, D), jnp.float32),
        jax.ShapeDtypeStruct((NC, NP, 16), jnp.float32),
    ),
    mesh=_mesh,
    compiler_params=pltpu.CompilerParams(use_tc_tiling_on_sc=False),
    scratch_types=[
        [pltpu.VMEM((2, 2, K), jnp.int32)] * 2,
        [pltpu.VMEM((K, D), jnp.float32)] * 2,
        pltpu.VMEM((K, 16), jnp.float32),
        pltpu.VMEM_SHARED((NP, D), jnp.float32),
        pltpu.VMEM_SHARED((NP, 16), jnp.float32),
        [pltpu.SemaphoreType.DMA] * 2,
        [pltpu.SemaphoreType.DMA] * 2,
    ],
)
def _sc_gather_sum(x_hbm, sd_hbm, zn_hbm, zd_hbm, ones_hbm,
                   neigh_out, deg_out,
                   idxb, bufs, ones_v, sneigh, sdeg, gsems, isems):
    c = lax.axis_index("c")
    s = lax.axis_index("s")
    wid = c * NS + s

    r0 = s * RPT
    pltpu.sync_copy(zn_hbm, sneigh.at[pl.ds(r0, RPT)])
    pltpu.sync_copy(zd_hbm, sdeg.at[pl.ds(r0, RPT)])

    pltpu.sync_copy(ones_hbm, ones_v)

    pltpu.sync_copy(sd_hbm.at[wid].at[pl.ds(0, 2)], idxb[0])
    pltpu.async_copy(sd_hbm.at[wid].at[pl.ds(2, 2)], idxb[1], isems[1])
    plsc.subcore_barrier()

    for i in range(2):
        pltpu.async_copy(x_hbm.at[idxb[0].at[i, 0]], bufs[i], gsems[i])

    def do_group(g, p, prefetch, load_next):
        if prefetch:
            pltpu.make_async_copy(sd_hbm.at[wid].at[pl.ds(0, 2)],
                                  idxb[1 - p], isems[1 - p]).wait()
        for i in range(2):
            pltpu.make_async_copy(x_hbm.at[idxb[p].at[i, 0]], bufs[i],
                                  gsems[i]).wait()
            pltpu.sync_copy(bufs[i], sneigh.at[idxb[p].at[i, 1]], add=True)
            pltpu.sync_copy(ones_v, sdeg.at[idxb[p].at[i, 1]], add=True)
            if prefetch:
                pltpu.async_copy(x_hbm.at[idxb[1 - p].at[i, 0]], bufs[i],
                                 gsems[i])
        if load_next:
            pltpu.async_copy(sd_hbm.at[wid].at[pl.ds((g + 2) * 2, 2)],
                             idxb[p], isems[p])

    def pair(t, carry):
        do_group(2 * t, 0, True, True)
        do_group(2 * t + 1, 1, True, True)
        return carry

    lax.fori_loop(0, NG // 2 - 1, pair, 0)
    do_group(NG - 2, 0, True, False)
    do_group(NG - 1, 1, False, False)

    plsc.subcore_barrier()

    pltpu.sync_copy(sneigh.at[pl.ds(r0, RPT)], neigh_out.at[c].at[pl.ds(r0, RPT)])
    pltpu.sync_copy(sdeg.at[pl.ds(r0, RPT)], deg_out.at[c].at[pl.ds(r0, RPT)])


_R = 2000


def _tc_body(x_ref, p0_ref, p1_ref, d0_ref, d1_ref, w1_ref, w2_ref, o_ref):
    deg = jnp.maximum(d0_ref[:, 0:1] + d1_ref[:, 0:1], 1.0)
    neigh = (p0_ref[...] + p1_ref[...]) / deg
    dn = (((1,), (1,)), ((), ()))
    o_ref[...] = (
        lax.dot_general(neigh, w1_ref[...], dn, preferred_element_type=jnp.float32)
        + lax.dot_general(x_ref[...], w2_ref[...], dn, preferred_element_type=jnp.float32)
    )


_tc_final = pl.pallas_call(
    _tc_body,
    grid=(N // _R,),
    in_specs=[
        pl.BlockSpec((_R, D), lambda i: (i, 0)),
        pl.BlockSpec((_R, D), lambda i: (i, 0)),
        pl.BlockSpec((_R, D), lambda i: (i, 0)),
        pl.BlockSpec((_R, 16), lambda i: (i, 0)),
        pl.BlockSpec((_R, 16), lambda i: (i, 0)),
        pl.BlockSpec((D, D), lambda i: (0, 0)),
        pl.BlockSpec((D, D), lambda i: (0, 0)),
    ],
    out_specs=pl.BlockSpec((_R, D), lambda i: (i, 0)),
    out_shape=jax.ShapeDtypeStruct((N, D), jnp.float32),
)


def kernel(x, edge_index, l, W1, W2):
    src = edge_index[0]
    dst = edge_index[1]
    src_p = jnp.concatenate([src, jnp.zeros((PAD,), jnp.int32)]).reshape(NW, CH, K)
    dst_p = jnp.concatenate([dst, jnp.full((PAD,), N, jnp.int32)]).reshape(NW, CH, K)
    sd = jnp.stack([src_p, dst_p], axis=2)
    zn = jnp.zeros((RPT, D), jnp.float32)
    zd = jnp.zeros((RPT, 16), jnp.float32)
    ones = jnp.ones((K, 16), jnp.float32)
    neigh2, deg2 = _sc_gather_sum(x, sd, zn, zd, ones)
    return _tc_final(x, neigh2[0, :N], neigh2[1, :N],
                     deg2[0, :N], deg2[1, :N], W1, W2)


# speedup vs baseline: 1.0189x; 1.0189x over previous
import functools

import jax
import jax.numpy as jnp
from jax import lax
from jax.experimental import pallas as pl
from jax.experimental.pallas import tpu as pltpu
from jax.experimental.pallas import tpu_sc as plsc

N = 10000
E = 320000
D = 128
NC = 2
NS = 16
NW = NC * NS
K = 128
CH = 80
EPW = CH * K
PAD = NW * EPW - E
NP = 10112
RPT = NP // NS
NG = CH // 2

_mesh = plsc.VectorSubcoreMesh(core_axis_name="c", subcore_axis_name="s")


@functools.partial(
    pl.kernel,
    out_type=(
        jax.ShapeDtypeStruct((NC, NP, D), jnp.float32),
        jax.ShapeDtypeStruct((NC, NP, 16), jnp.float32),
    ),
    mesh=_mesh,
    compiler_params=pltpu.CompilerParams(use_tc_tiling_on_sc=False),
    scratch_types=[
        [pltpu.VMEM((2, 2, K), jnp.int32)] * 2,
        [pltpu.VMEM((K, D), jnp.float32)] * 2,
        pltpu.VMEM((K, 16), jnp.float32),
        pltpu.VMEM_SHARED((NP, D), jnp.float32),
        pltpu.VMEM_SHARED((NP, 16), jnp.float32),
        [pltpu.SemaphoreType.DMA] * 2,
        [pltpu.SemaphoreType.DMA] * 2,
    ],
)
def _sc_gather_sum(x_hbm, sd_hbm, zn_hbm, zd_hbm, ones_hbm,
                   neigh_out, deg_out,
                   idxb, bufs, ones_v, sneigh, sdeg, gsems, isems):
    c = lax.axis_index("c")
    s = lax.axis_index("s")
    wid = c * NS + s

    r0 = s * RPT
    pltpu.sync_copy(zn_hbm, sneigh.at[pl.ds(r0, RPT)])
    pltpu.sync_copy(zd_hbm, sdeg.at[pl.ds(r0, RPT)])

    pltpu.sync_copy(ones_hbm, ones_v)

    pltpu.sync_copy(sd_hbm.at[wid].at[pl.ds(0, 2)], idxb[0])
    pltpu.async_copy(sd_hbm.at[wid].at[pl.ds(2, 2)], idxb[1], isems[1])
    plsc.subcore_barrier()

    for i in range(2):
        pltpu.async_copy(x_hbm.at[idxb[0].at[i, 0]], bufs[i], gsems[i])

    def do_group(g, p, prefetch, load_next):
        if prefetch:
            pltpu.make_async_copy(sd_hbm.at[wid].at[pl.ds(0, 2)],
                                  idxb[1 - p], isems[1 - p]).wait()
        for i in range(2):
            pltpu.make_async_copy(x_hbm.at[idxb[p].at[i, 0]], bufs[i],
                                  gsems[i]).wait()
            pltpu.sync_copy(bufs[i], sneigh.at[idxb[p].at[i, 1]], add=True)
            pltpu.sync_copy(ones_v, sdeg.at[idxb[p].at[i, 1]], add=True)
            if prefetch:
                pltpu.async_copy(x_hbm.at[idxb[1 - p].at[i, 0]], bufs[i],
                                 gsems[i])
        if load_next:
            pltpu.async_copy(sd_hbm.at[wid].at[pl.ds((g + 2) * 2, 2)],
                             idxb[p], isems[p])

    def pair(t, carry):
        do_group(2 * t, 0, True, True)
        do_group(2 * t + 1, 1, True, True)
        return carry

    lax.fori_loop(0, NG // 2 - 1, pair, 0)
    do_group(NG - 2, 0, True, False)
    do_group(NG - 1, 1, False, False)

    plsc.subcore_barrier()

    pltpu.sync_copy(sneigh.at[pl.ds(r0, RPT)], neigh_out.at[c].at[pl.ds(r0, RPT)])
    pltpu.sync_copy(sdeg.at[pl.ds(r0, RPT)], deg_out.at[c].at[pl.ds(r0, RPT)])


_R = 2000


def _tc_body(x_ref, p0_ref, p1_ref, d0_ref, d1_ref, w1_ref, w2_ref, o_ref):
    deg = jnp.maximum(d0_ref[:, 0:1] + d1_ref[:, 0:1], 1.0)
    neigh = (p0_ref[...] + p1_ref[...]) / deg
    dn = (((1,), (1,)), ((), ()))
    o_ref[...] = (
        lax.dot_general(neigh, w1_ref[...], dn, preferred_element_type=jnp.float32)
        + lax.dot_general(x_ref[...], w2_ref[...], dn, preferred_element_type=jnp.float32)
    )


_tc_final = pl.pallas_call(
    _tc_body,
    grid=(N // _R,),
    in_specs=[
        pl.BlockSpec((_R, D), lambda i: (i, 0)),
        pl.BlockSpec((_R, D), lambda i: (i, 0)),
        pl.BlockSpec((_R, D), lambda i: (i, 0)),
        pl.BlockSpec((_R, 16), lambda i: (i, 0)),
        pl.BlockSpec((_R, 16), lambda i: (i, 0)),
        pl.BlockSpec((D, D), lambda i: (0, 0)),
        pl.BlockSpec((D, D), lambda i: (0, 0)),
    ],
    out_specs=pl.BlockSpec((_R, D), lambda i: (i, 0)),
    out_shape=jax.ShapeDtypeStruct((N, D), jnp.float32),
)


def kernel(x, edge_index, l, W1, W2):
    src = edge_index[0]
    dst = edge_index[1]
    src_p = jnp.concatenate([src, jnp.zeros((PAD,), jnp.int32)]).reshape(NW, CH, K)
    dst_p = jnp.concatenate([dst, jnp.full((PAD,), N, jnp.int32)]).reshape(NW, CH, K)
    sd = jnp.stack([src_p, dst_p], axis=2)
    zn = jnp.zeros((RPT, D), jnp.float32)
    zd = jnp.zeros((RPT, 16), jnp.float32)
    ones = jnp.ones((K, 16), jnp.float32)
    neigh2, deg2 = _sc_gather_sum(x, sd, zn, zd, ones)
    return _tc_final(x, neigh2[0, :N], neigh2[1, :N],
                     deg2[0, :N], deg2[1, :N], W1, W2)


# speedup vs baseline: 1.3446x; 1.3197x over previous
import functools

import jax
import jax.numpy as jnp
from jax import lax
from jax.experimental import pallas as pl
from jax.experimental.pallas import tpu as pltpu
from jax.experimental.pallas import tpu_sc as plsc

N = 10000
E = 320000
D = 128
DH = D // 2
NC = 2
NS = 16
K = 128
CH = 160
EPW = CH * K
PAD = NS * EPW - E
NP = 10112
RPT = NP // NS

_mesh = plsc.VectorSubcoreMesh(core_axis_name="c", subcore_axis_name="s")


@functools.partial(
    pl.kernel,
    out_type=(
        jax.ShapeDtypeStruct((NC, NP, DH), jnp.float32),
        jax.ShapeDtypeStruct((NC, NP, 16), jnp.float32),
    ),
    mesh=_mesh,
    compiler_params=pltpu.CompilerParams(use_tc_tiling_on_sc=False),
    scratch_types=[
        pltpu.VMEM((CH, K), jnp.int32),
        pltpu.VMEM((CH, K), jnp.int32),
        [pltpu.VMEM((K, DH), jnp.float32)] * 4,
        pltpu.VMEM((K, 16), jnp.float32),
        pltpu.VMEM_SHARED((NP, DH), jnp.float32),
        pltpu.VMEM_SHARED((NP, 16), jnp.float32),
        [pltpu.SemaphoreType.DMA] * 4,
    ],
)
def _sc_gather_sum(xi_hbm, src0_hbm, src1_hbm, dst_hbm, zn_hbm, zd_hbm, ones_hbm,
                   neigh_out, deg_out,
                   src_v, dst_v, bufs, ones_v, sneigh, sdeg, sems):
    c = lax.axis_index("c")
    s = lax.axis_index("s")

    r0 = s * RPT
    pltpu.sync_copy(zn_hbm, sneigh.at[pl.ds(r0, RPT)])
    pltpu.sync_copy(zd_hbm, sdeg.at[pl.ds(r0, RPT)])
    pltpu.sync_copy(ones_hbm, ones_v)
    pltpu.sync_copy(dst_hbm.at[s], dst_v)

    @pl.when(c == 0)
    def _():
        pltpu.sync_copy(src0_hbm.at[s], src_v)

    @pl.when(c == 1)
    def _():
        pltpu.sync_copy(src1_hbm.at[s], src_v)

    plsc.subcore_barrier()

    NB = 4
    GRP = CH // NB

    for b in range(NB):
        pltpu.async_copy(xi_hbm.at[src_v.at[b]], bufs[b], sems[b])

    def process(k, prefetch):
        for b in range(NB):
            j = k * NB + b
            pltpu.make_async_copy(xi_hbm.at[src_v.at[j]], bufs[b], sems[b]).wait()
            pltpu.sync_copy(bufs[b], sneigh.at[dst_v.at[j]], add=True)
            pltpu.sync_copy(ones_v, sdeg.at[dst_v.at[j]], add=True)
            if prefetch:
                pltpu.async_copy(xi_hbm.at[src_v.at[j + NB]], bufs[b], sems[b])

    def group(k, carry):
        process(k, True)
        return carry

    lax.fori_loop(0, GRP - 1, group, 0)
    process(GRP - 1, False)

    plsc.subcore_barrier()

    pltpu.sync_copy(sneigh.at[pl.ds(r0, RPT)], neigh_out.at[c].at[pl.ds(r0, RPT)])
    pltpu.sync_copy(sdeg.at[pl.ds(r0, RPT)], deg_out.at[c].at[pl.ds(r0, RPT)])


_R = 2000


def _tc_body(x_ref, p0_ref, p1_ref, d0_ref, w1_ref, w2_ref, o_ref):
    deg = jnp.maximum(d0_ref[:, 0:1], 1.0)
    neigh = jnp.concatenate([p0_ref[...], p1_ref[...]], axis=1) / deg
    dn = (((1,), (1,)), ((), ()))
    o_ref[...] = (
        lax.dot_general(neigh, w1_ref[...], dn, preferred_element_type=jnp.float32)
        + lax.dot_general(x_ref[...], w2_ref[...], dn, preferred_element_type=jnp.float32)
    )


_tc_final = pl.pallas_call(
    _tc_body,
    grid=(N // _R,),
    in_specs=[
        pl.BlockSpec((_R, D), lambda i: (i, 0)),
        pl.BlockSpec((_R, DH), lambda i: (i, 0)),
        pl.BlockSpec((_R, DH), lambda i: (i, 0)),
        pl.BlockSpec((_R, 16), lambda i: (i, 0)),
        pl.BlockSpec((D, D), lambda i: (0, 0)),
        pl.BlockSpec((D, D), lambda i: (0, 0)),
    ],
    out_specs=pl.BlockSpec((_R, D), lambda i: (i, 0)),
    out_shape=jax.ShapeDtypeStruct((N, D), jnp.float32),
)


def kernel(x, edge_index, l, W1, W2):
    src = edge_index[0]
    dst = edge_index[1]
    xi = x.reshape(2 * N, DH)
    src_pad = jnp.concatenate([src, jnp.zeros((PAD,), jnp.int32)])
    src0 = (2 * src_pad).reshape(NS, CH, K)
    src1 = (2 * src_pad + 1).reshape(NS, CH, K)
    dst_p = jnp.concatenate([dst, jnp.full((PAD,), N, jnp.int32)]).reshape(NS, CH, K)
    zn = jnp.zeros((RPT, DH), jnp.float32)
    zd = jnp.zeros((RPT, 16), jnp.float32)
    ones = jnp.ones((K, 16), jnp.float32)
    neigh2, deg2 = _sc_gather_sum(xi, src0, src1, dst_p, zn, zd, ones)
    return _tc_final(x, neigh2[0, :N], neigh2[1, :N], deg2[0, :N], W1, W2)


# speedup vs baseline: 1.5365x; 1.1427x over previous
import functools

import jax
import jax.numpy as jnp
from jax import lax
from jax.experimental import pallas as pl
from jax.experimental.pallas import tpu as pltpu
from jax.experimental.pallas import tpu_sc as plsc

N = 10000
E = 320000
D = 128
DH = D // 2
NC = 2
NS = 16
K = 128
CH = 160
EPW = CH * K
PAD = NS * EPW - E
NP = 10112
RPT = NP // NS

_mesh = plsc.VectorSubcoreMesh(core_axis_name="c", subcore_axis_name="s")


@functools.partial(
    pl.kernel,
    out_type=(
        jax.ShapeDtypeStruct((NC, NP, DH), jnp.float32),
        jax.ShapeDtypeStruct((NC, NP, 16), jnp.float32),
    ),
    mesh=_mesh,
    compiler_params=pltpu.CompilerParams(use_tc_tiling_on_sc=False),
    scratch_types=[
        pltpu.VMEM((CH, K), jnp.int32),
        pltpu.VMEM((CH, K), jnp.int32),
        [pltpu.VMEM((K, DH), jnp.float32)] * 4,
        pltpu.VMEM((K, 16), jnp.float32),
        pltpu.VMEM_SHARED((NP, DH), jnp.float32),
        pltpu.VMEM_SHARED((NP, 16), jnp.float32),
        [pltpu.SemaphoreType.DMA] * 4,
    ],
)
def _sc_gather_sum(xl_hbm, xr_hbm, src_hbm, dst_hbm, zn_hbm, zd_hbm, ones_hbm,
                   neigh_out, deg_out,
                   src_v, dst_v, bufs, ones_v, sneigh, sdeg, sems):
    c = lax.axis_index("c")
    s = lax.axis_index("s")

    r0 = s * RPT
    pltpu.sync_copy(zn_hbm, sneigh.at[pl.ds(r0, RPT)])
    pltpu.sync_copy(zd_hbm, sdeg.at[pl.ds(r0, RPT)])
    pltpu.sync_copy(ones_hbm, ones_v)
    pltpu.sync_copy(dst_hbm.at[s], dst_v)
    pltpu.sync_copy(src_hbm.at[s], src_v)

    plsc.subcore_barrier()

    NB = 4
    GRP = CH // NB

    def do_chunks(x_hbm):
        for b in range(NB):
            pltpu.async_copy(x_hbm.at[src_v.at[b]], bufs[b], sems[b])

        def process(k, prefetch):
            for b in range(NB):
                j = k * NB + b
                pltpu.make_async_copy(x_hbm.at[src_v.at[j]], bufs[b], sems[b]).wait()
                pltpu.sync_copy(bufs[b], sneigh.at[dst_v.at[j]], add=True)
                pltpu.sync_copy(ones_v, sdeg.at[dst_v.at[j]], add=True)
                if prefetch:
                    pltpu.async_copy(x_hbm.at[src_v.at[j + NB]], bufs[b], sems[b])

        def group(k, carry):
            process(k, True)
            return carry

        lax.fori_loop(0, GRP - 1, group, 0)
        process(GRP - 1, False)

    @pl.when(c == 0)
    def _():
        do_chunks(xl_hbm)

    @pl.when(c == 1)
    def _():
        do_chunks(xr_hbm)

    plsc.subcore_barrier()

    pltpu.sync_copy(sneigh.at[pl.ds(r0, RPT)], neigh_out.at[c].at[pl.ds(r0, RPT)])
    pltpu.sync_copy(sdeg.at[pl.ds(r0, RPT)], deg_out.at[c].at[pl.ds(r0, RPT)])


_R = 2000


def _tc_body(x_ref, p0_ref, p1_ref, d0_ref, w1_ref, w2_ref, o_ref):
    deg = jnp.maximum(d0_ref[:, 0:1], 1.0)
    neigh = jnp.concatenate([p0_ref[...], p1_ref[...]], axis=1) / deg
    dn = (((1,), (1,)), ((), ()))
    o_ref[...] = (
        lax.dot_general(neigh, w1_ref[...], dn, preferred_element_type=jnp.float32)
        + lax.dot_general(x_ref[...], w2_ref[...], dn, preferred_element_type=jnp.float32)
    )


_tc_final = pl.pallas_call(
    _tc_body,
    grid=(N // _R,),
    in_specs=[
        pl.BlockSpec((_R, D), lambda i: (i, 0)),
        pl.BlockSpec((_R, DH), lambda i: (i, 0)),
        pl.BlockSpec((_R, DH), lambda i: (i, 0)),
        pl.BlockSpec((_R, 16), lambda i: (i, 0)),
        pl.BlockSpec((D, D), lambda i: (0, 0)),
        pl.BlockSpec((D, D), lambda i: (0, 0)),
    ],
    out_specs=pl.BlockSpec((_R, D), lambda i: (i, 0)),
    out_shape=jax.ShapeDtypeStruct((N, D), jnp.float32),
)


def kernel(x, edge_index, l, W1, W2):
    src = edge_index[0]
    dst = edge_index[1]
    xl = x[:, :DH]
    xr = x[:, DH:]
    src_p = jnp.concatenate([src, jnp.zeros((PAD,), jnp.int32)]).reshape(NS, CH, K)
    dst_p = jnp.concatenate([dst, jnp.full((PAD,), N, jnp.int32)]).reshape(NS, CH, K)
    zn = jnp.zeros((RPT, DH), jnp.float32)
    zd = jnp.zeros((RPT, 16), jnp.float32)
    ones = jnp.ones((K, 16), jnp.float32)
    neigh2, deg2 = _sc_gather_sum(xl, xr, src_p, dst_p, zn, zd, ones)
    return _tc_final(x, neigh2[0, :N], neigh2[1, :N], deg2[0, :N], W1, W2)
